# baseline (speedup 1.0000x reference)
import functools

import jax
import jax.numpy as jnp
from jax import lax
from jax.experimental import pallas as pl
from jax.experimental.pallas import tpu as pltpu
from jax.experimental.pallas import tpu_sc as plsc

_MIN_NORM = 1e-15
_PROJ_EPS = 4e-3

_NC = 2
_NS = 16
_NW = _NC * _NS
_L = 16

_CE = 8000


def _artanh(v):
    v = jnp.clip(v, -1.0 + 1e-7, 1.0 - 1e-7)
    return 0.5 * (jnp.log1p(v) - jnp.log1p(-v))


def _logmap0_packed_body(x_ref, o_ref):
    x = x_ref[...]
    nrm = jnp.sqrt(jnp.sum(x * x, axis=1, keepdims=True))
    nrm = jnp.maximum(nrm, _MIN_NORM)
    scale = _artanh(nrm) / nrm
    xt_t = (x * scale).T
    half = xt_t.shape[0] // 2
    lo = lax.bitcast_convert_type(
        xt_t[:half].astype(jnp.bfloat16), jnp.uint16
    ).astype(jnp.int32)
    hi = lax.bitcast_convert_type(
        xt_t[half:].astype(jnp.bfloat16), jnp.uint16
    ).astype(jnp.int32)
    o_ref[...] = lo | (hi << 16)


def _expmap0_proj_body(a_ref, o_ref):
    u = a_ref[...].T
    u_norm = jnp.maximum(
        jnp.sqrt(jnp.sum(u * u, axis=1, keepdims=True)), _MIN_NORM
    )
    gamma = jnp.tanh(u_norm) * u / u_norm
    g_norm = jnp.maximum(
        jnp.sqrt(jnp.sum(gamma * gamma, axis=1, keepdims=True)), _MIN_NORM
    )
    maxnorm = 1.0 - _PROJ_EPS
    projected = gamma / g_norm * maxnorm
    o_ref[...] = jnp.where(g_norm > maxnorm, projected, gamma)


def _make_spmm(n, d, e):
    half = d // 2
    ppt = half // _NW
    fpt = 2 * ppt
    pk_words = ppt * n
    acc_words = fpt * n
    n_chunks = e // _CE
    gpc = _CE // _L
    assert n <= 16384
    cw = 2 * _CE
    assert n_chunks % 2 == 0

    mesh = plsc.VectorSubcoreMesh(core_axis_name="c", subcore_axis_name="s")

    @functools.partial(
        pl.kernel,
        mesh=mesh,
        compiler_params=pltpu.CompilerParams(needs_layout_passes=False),
        out_type=jax.ShapeDtypeStruct((d * n,), jnp.float32),
        scratch_types=[
            pltpu.VMEM((pk_words,), jnp.int32),
            pltpu.VMEM((acc_words,), jnp.float32),
            pltpu.VMEM((cw + _L,), jnp.int32),
            pltpu.VMEM((cw + _L,), jnp.int32),
            pltpu.SemaphoreType.DMA,
            pltpu.SemaphoreType.DMA,
            pltpu.SemaphoreType.DMA,
        ],
    )
    def spmm(pk_hbm, edges_hbm, out_hbm, pk_v, acc_v, eb0, eb1, sem0, sem1, semx):
        wid = lax.axis_index("s") * _NC + lax.axis_index("c")
        pk_base = pl.multiple_of(wid * pk_words, 8)
        xt_copy = pltpu.async_copy(pk_hbm.at[pl.ds(pk_base, pk_words)], pk_v, semx)
        pltpu.async_copy(edges_hbm.at[pl.ds(0, cw)], eb0.at[pl.ds(0, cw)], sem0)

        zeros = jnp.zeros((_L,), jnp.float32)

        def zero_body(i, carry):
            acc_v[pl.ds(i * _L, _L)] = zeros
            return carry

        lax.fori_loop(0, acc_words // _L, zero_body, 0, unroll=8)
        xt_copy.wait()

        ebufs = (eb0, eb1)
        sems = (sem0, sem1)
        himask = jnp.int32(-65536)

        def outer_body(o, carry):
            for b in range(2):
                c = o * 2 + b
                eb = ebufs[b]
                pltpu.make_async_copy(
                    edges_hbm.at[pl.ds(0, cw)], eb.at[pl.ds(0, cw)], sems[b]
                ).wait()
                @pl.when(c + 1 < n_chunks)
                def _():
                    off = pl.multiple_of((c + 1) * cw, 8)
                    pltpu.async_copy(
                        edges_hbm.at[pl.ds(off, cw)],
                        ebufs[1 - b].at[pl.ds(0, cw)],
                        sems[1 - b],
                    )

                def load_grp(g):
                    gb = g * _L
                    pair = eb[pl.ds(gb, _L)]
                    s = pair & 16383
                    dd = lax.shift_right_logical(pair, 14)
                    w = plsc.bitcast(eb[pl.ds(_CE + gb, _L)], jnp.float32)
                    return s, dd, w

                def group_body(g, carry):
                    s, dd, w = carry
                    nxt = load_grp(g + 1)
                    pk = [
                        plsc.load_gather(pk_v, [s + p * n]) for p in range(ppt)
                    ]
                    for p in range(ppt):
                        lo = plsc.bitcast(pk[p] << 16, jnp.float32)
                        hi = plsc.bitcast(pk[p] & himask, jnp.float32)
                        plsc.addupdate_scatter(acc_v, [dd + p * n], lo * w)
                        plsc.addupdate_scatter(
                            acc_v, [dd + (ppt + p) * n], hi * w
                        )
                    return nxt

                lax.fori_loop(0, gpc, group_body, load_grp(0), unroll=10)
            return carry

        lax.fori_loop(0, n_chunks // 2, outer_body, 0)
        lo_base = pl.multiple_of(wid * ppt * n, 8)
        hi_base = pl.multiple_of((half + wid * ppt) * n, 8)
        pltpu.sync_copy(
            acc_v.at[pl.ds(0, ppt * n)], out_hbm.at[pl.ds(lo_base, ppt * n)]
        )
        pltpu.sync_copy(
            acc_v.at[pl.ds(ppt * n, ppt * n)],
            out_hbm.at[pl.ds(hi_base, ppt * n)],
        )

    return spmm


@jax.jit
def kernel(x, edge_index, edge_weight):
    n, d = x.shape
    e = edge_index.shape[1]
    src = edge_index[0]
    dst = edge_index[1]

    pk_t = pl.pallas_call(
        _logmap0_packed_body,
        out_shape=jax.ShapeDtypeStruct((d // 2, n), jnp.int32),
    )(x)

    nc = e // _CE
    w_i = lax.bitcast_convert_type(edge_weight, jnp.int32)
    pair = src + dst * 16384
    edges_packed = jnp.concatenate(
        [pair.reshape(nc, _CE), w_i.reshape(nc, _CE)], axis=1
    ).reshape(-1)

    spmm = _make_spmm(n, d, e)
    support_t = spmm(pk_t.reshape(d // 2 * n), edges_packed)

    out = pl.pallas_call(
        _expmap0_proj_body,
        out_shape=jax.ShapeDtypeStruct((n, d), jnp.float32),
    )(support_t.reshape(d, n))

    return out


# speedup vs baseline: 1.2454x; 1.2454x over previous
import functools

import jax
import jax.numpy as jnp
from jax import lax
from jax.experimental import pallas as pl
from jax.experimental.pallas import tpu as pltpu
from jax.experimental.pallas import tpu_sc as plsc

_MIN_NORM = 1e-15
_PROJ_EPS = 4e-3

_NC = 2
_NS = 16
_NW = _NC * _NS
_L = 16

_CE = 8000


def _artanh(v):
    v = jnp.clip(v, -1.0 + 1e-7, 1.0 - 1e-7)
    return 0.5 * (jnp.log1p(v) - jnp.log1p(-v))


def _logmap0_packed_body(x_ref, o_ref):
    x = x_ref[...]
    nrm = jnp.sqrt(jnp.sum(x * x, axis=1, keepdims=True))
    nrm = jnp.maximum(nrm, _MIN_NORM)
    scale = _artanh(nrm) / nrm
    xt_t = (x * scale).T
    half = xt_t.shape[0] // 2
    lo = lax.bitcast_convert_type(
        xt_t[:half].astype(jnp.bfloat16), jnp.uint16
    ).astype(jnp.int32)
    hi = lax.bitcast_convert_type(
        xt_t[half:].astype(jnp.bfloat16), jnp.uint16
    ).astype(jnp.int32)
    o_ref[...] = lo | (hi << 16)


def _expmap0_proj_body(a_ref, o_ref):
    u = a_ref[...].T
    u_norm = jnp.maximum(
        jnp.sqrt(jnp.sum(u * u, axis=1, keepdims=True)), _MIN_NORM
    )
    gamma = jnp.tanh(u_norm) * u / u_norm
    g_norm = jnp.maximum(
        jnp.sqrt(jnp.sum(gamma * gamma, axis=1, keepdims=True)), _MIN_NORM
    )
    maxnorm = 1.0 - _PROJ_EPS
    projected = gamma / g_norm * maxnorm
    o_ref[...] = jnp.where(g_norm > maxnorm, projected, gamma)


def _make_spmm(n, d, e):
    half = d // 2
    ppt = half // _NW
    fpt = 2 * ppt
    pk_words = ppt * n
    acc_words = fpt * n
    n_chunks = e // _CE
    gpc = _CE // _L
    assert n <= 16384
    cw = 2 * _CE
    assert n_chunks % 2 == 0

    mesh = plsc.VectorSubcoreMesh(core_axis_name="c", subcore_axis_name="s")

    @functools.partial(
        pl.kernel,
        mesh=mesh,
        compiler_params=pltpu.CompilerParams(needs_layout_passes=False),
        out_type=jax.ShapeDtypeStruct((d * n,), jnp.float32),
        scratch_types=[
            pltpu.VMEM((pk_words,), jnp.int32),
            pltpu.VMEM((acc_words,), jnp.float32),
            pltpu.VMEM((cw + _L,), jnp.int32),
            pltpu.VMEM((cw + _L,), jnp.int32),
            pltpu.SemaphoreType.DMA,
            pltpu.SemaphoreType.DMA,
            pltpu.SemaphoreType.DMA,
        ],
    )
    def spmm(pk_hbm, edges_hbm, out_hbm, pk_v, acc_v, eb0, eb1, sem0, sem1, semx):
        wid = lax.axis_index("s") * _NC + lax.axis_index("c")
        pk_base = pl.multiple_of(wid * pk_words, 8)
        xt_copy = pltpu.async_copy(pk_hbm.at[pl.ds(pk_base, pk_words)], pk_v, semx)
        pltpu.async_copy(edges_hbm.at[pl.ds(0, cw)], eb0.at[pl.ds(0, cw)], sem0)

        zeros = jnp.zeros((_L,), jnp.float32)

        def zero_body(i, carry):
            acc_v[pl.ds(i * _L, _L)] = zeros
            return carry

        lax.fori_loop(0, acc_words // _L, zero_body, 0, unroll=8)
        xt_copy.wait()

        ebufs = (eb0, eb1)
        sems = (sem0, sem1)
        himask = jnp.int32(-65536)

        def outer_body(o, carry):
            for b in range(2):
                c = o * 2 + b
                eb = ebufs[b]
                pltpu.make_async_copy(
                    edges_hbm.at[pl.ds(0, cw)], eb.at[pl.ds(0, cw)], sems[b]
                ).wait()
                @pl.when(c + 1 < n_chunks)
                def _():
                    off = pl.multiple_of((c + 1) * cw, 8)
                    pltpu.async_copy(
                        edges_hbm.at[pl.ds(off, cw)],
                        ebufs[1 - b].at[pl.ds(0, cw)],
                        sems[1 - b],
                    )

                @plsc.parallel_loop(0, gpc, unroll=8)
                def _(g):
                    gb = g * _L
                    pair = eb[pl.ds(gb, _L)]
                    s = pair & 16383
                    dd = lax.shift_right_logical(pair, 14)
                    w = plsc.bitcast(eb[pl.ds(_CE + gb, _L)], jnp.float32)
                    pk = [
                        plsc.load_gather(pk_v, [s + p * n]) for p in range(ppt)
                    ]
                    for p in range(ppt):
                        lo = plsc.bitcast(pk[p] << 16, jnp.float32)
                        hi = plsc.bitcast(pk[p] & himask, jnp.float32)
                        plsc.addupdate_scatter(acc_v, [dd + p * n], lo * w)
                        plsc.addupdate_scatter(
                            acc_v, [dd + (ppt + p) * n], hi * w
                        )
            return carry

        lax.fori_loop(0, n_chunks // 2, outer_body, 0)
        lo_base = pl.multiple_of(wid * ppt * n, 8)
        hi_base = pl.multiple_of((half + wid * ppt) * n, 8)
        pltpu.sync_copy(
            acc_v.at[pl.ds(0, ppt * n)], out_hbm.at[pl.ds(lo_base, ppt * n)]
        )
        pltpu.sync_copy(
            acc_v.at[pl.ds(ppt * n, ppt * n)],
            out_hbm.at[pl.ds(hi_base, ppt * n)],
        )

    return spmm


@jax.jit
def kernel(x, edge_index, edge_weight):
    n, d = x.shape
    e = edge_index.shape[1]
    src = edge_index[0]
    dst = edge_index[1]

    pk_t = pl.pallas_call(
        _logmap0_packed_body,
        out_shape=jax.ShapeDtypeStruct((d // 2, n), jnp.int32),
    )(x)

    nc = e // _CE
    w_i = lax.bitcast_convert_type(edge_weight, jnp.int32)
    pair = src + dst * 16384
    edges_packed = jnp.concatenate(
        [pair.reshape(nc, _CE), w_i.reshape(nc, _CE)], axis=1
    ).reshape(-1)

    spmm = _make_spmm(n, d, e)
    support_t = spmm(pk_t.reshape(d // 2 * n), edges_packed)

    out = pl.pallas_call(
        _expmap0_proj_body,
        out_shape=jax.ShapeDtypeStruct((n, d), jnp.float32),
    )(support_t.reshape(d, n))

    return out


# speedup vs baseline: 1.3365x; 1.0731x over previous
import functools

import jax
import jax.numpy as jnp
from jax import lax
from jax.experimental import pallas as pl
from jax.experimental.pallas import tpu as pltpu
from jax.experimental.pallas import tpu_sc as plsc

_MIN_NORM = 1e-15
_PROJ_EPS = 4e-3

_NC = 2
_NS = 16
_NW = _NC * _NS
_L = 16

_CE = 8000


def _artanh(v):
    v = jnp.clip(v, -1.0 + 1e-7, 1.0 - 1e-7)
    return 0.5 * (jnp.log1p(v) - jnp.log1p(-v))


def _logmap0_packed_body(x_ref, ei_ref, o_ref, oe_ref):
    x = x_ref[...]
    nrm = jnp.sqrt(jnp.sum(x * x, axis=1, keepdims=True))
    nrm = jnp.maximum(nrm, _MIN_NORM)
    scale = _artanh(nrm) / nrm
    xt_t = (x * scale).T
    half = xt_t.shape[0] // 2
    lo = lax.bitcast_convert_type(
        xt_t[:half].astype(jnp.bfloat16), jnp.uint16
    ).astype(jnp.int32)
    hi = lax.bitcast_convert_type(
        xt_t[half:].astype(jnp.bfloat16), jnp.uint16
    ).astype(jnp.int32)
    o_ref[...] = lo | (hi << 16)
    oe_ref[...] = ei_ref[0, :] + ei_ref[1, :] * 16384


def _expmap0_proj_body(a_ref, o_ref):
    u = a_ref[...].T
    u_norm = jnp.maximum(
        jnp.sqrt(jnp.sum(u * u, axis=1, keepdims=True)), _MIN_NORM
    )
    gamma = jnp.tanh(u_norm) * u / u_norm
    g_norm = jnp.maximum(
        jnp.sqrt(jnp.sum(gamma * gamma, axis=1, keepdims=True)), _MIN_NORM
    )
    maxnorm = 1.0 - _PROJ_EPS
    projected = gamma / g_norm * maxnorm
    o_ref[...] = jnp.where(g_norm > maxnorm, projected, gamma)


def _make_spmm(n, d, e):
    half = d // 2
    ppt = half // _NW
    fpt = 2 * ppt
    pk_words = ppt * n
    acc_words = fpt * n
    n_chunks = e // _CE
    gpc = _CE // _L
    assert n <= 16384
    assert n_chunks % 2 == 0

    mesh = plsc.VectorSubcoreMesh(core_axis_name="c", subcore_axis_name="s")

    @functools.partial(
        pl.kernel,
        mesh=mesh,
        compiler_params=pltpu.CompilerParams(needs_layout_passes=False),
        out_type=jax.ShapeDtypeStruct((d * n,), jnp.float32),
        scratch_types=[
            pltpu.VMEM((pk_words,), jnp.int32),
            pltpu.VMEM((acc_words,), jnp.float32),
            pltpu.VMEM((_CE + _L,), jnp.int32),
            pltpu.VMEM((_CE + _L,), jnp.int32),
            pltpu.VMEM((_CE + _L,), jnp.float32),
            pltpu.VMEM((_CE + _L,), jnp.float32),
            pltpu.SemaphoreType.DMA,
            pltpu.SemaphoreType.DMA,
            pltpu.SemaphoreType.DMA,
        ],
    )
    def spmm(
        pk_hbm,
        pair_hbm,
        w_hbm,
        out_hbm,
        pk_v,
        acc_v,
        ep0,
        ep1,
        ew0,
        ew1,
        sem0,
        sem1,
        semx,
    ):
        wid = lax.axis_index("s") * _NC + lax.axis_index("c")
        pk_base = pl.multiple_of(wid * pk_words, 8)
        xt_copy = pltpu.async_copy(pk_hbm.at[pl.ds(pk_base, pk_words)], pk_v, semx)
        pltpu.async_copy(pair_hbm.at[pl.ds(0, _CE)], ep0.at[pl.ds(0, _CE)], sem0)
        pltpu.async_copy(w_hbm.at[pl.ds(0, _CE)], ew0.at[pl.ds(0, _CE)], sem0)

        zeros = jnp.zeros((_L,), jnp.float32)

        def zero_body(i, carry):
            acc_v[pl.ds(i * _L, _L)] = zeros
            return carry

        lax.fori_loop(0, acc_words // _L, zero_body, 0, unroll=8)
        xt_copy.wait()

        pbufs = (ep0, ep1)
        wbufs = (ew0, ew1)
        sems = (sem0, sem1)
        himask = jnp.int32(-65536)

        def outer_body(o, carry):
            for b in range(2):
                c = o * 2 + b
                ep = pbufs[b]
                ew = wbufs[b]
                pltpu.make_async_copy(
                    pair_hbm.at[pl.ds(0, _CE)], ep.at[pl.ds(0, _CE)], sems[b]
                ).wait()
                pltpu.make_async_copy(
                    w_hbm.at[pl.ds(0, _CE)], ew.at[pl.ds(0, _CE)], sems[b]
                ).wait()
                @pl.when(c + 1 < n_chunks)
                def _():
                    off = pl.multiple_of((c + 1) * _CE, 8)
                    pltpu.async_copy(
                        pair_hbm.at[pl.ds(off, _CE)],
                        pbufs[1 - b].at[pl.ds(0, _CE)],
                        sems[1 - b],
                    )
                    pltpu.async_copy(
                        w_hbm.at[pl.ds(off, _CE)],
                        wbufs[1 - b].at[pl.ds(0, _CE)],
                        sems[1 - b],
                    )

                @plsc.parallel_loop(0, gpc, unroll=8)
                def _(g):
                    gb = g * _L
                    pair = ep[pl.ds(gb, _L)]
                    s = pair & 16383
                    dd = lax.shift_right_logical(pair, 14)
                    w = ew[pl.ds(gb, _L)]
                    pk = [
                        plsc.load_gather(pk_v, [s + p * n]) for p in range(ppt)
                    ]
                    for p in range(ppt):
                        lo = plsc.bitcast(pk[p] << 16, jnp.float32)
                        hi = plsc.bitcast(pk[p] & himask, jnp.float32)
                        plsc.addupdate_scatter(acc_v, [dd + p * n], lo * w)
                        plsc.addupdate_scatter(
                            acc_v, [dd + (ppt + p) * n], hi * w
                        )
            return carry

        lax.fori_loop(0, n_chunks // 2, outer_body, 0)
        lo_base = pl.multiple_of(wid * ppt * n, 8)
        hi_base = pl.multiple_of((half + wid * ppt) * n, 8)
        pltpu.sync_copy(
            acc_v.at[pl.ds(0, ppt * n)], out_hbm.at[pl.ds(lo_base, ppt * n)]
        )
        pltpu.sync_copy(
            acc_v.at[pl.ds(ppt * n, ppt * n)],
            out_hbm.at[pl.ds(hi_base, ppt * n)],
        )

    return spmm


@jax.jit
def kernel(x, edge_index, edge_weight):
    n, d = x.shape
    e = edge_index.shape[1]

    pk_t, pair = pl.pallas_call(
        _logmap0_packed_body,
        out_shape=(
            jax.ShapeDtypeStruct((d // 2, n), jnp.int32),
            jax.ShapeDtypeStruct((e,), jnp.int32),
        ),
    )(x, edge_index)

    spmm = _make_spmm(n, d, e)
    support_t = spmm(pk_t.reshape(d // 2 * n), pair, edge_weight)

    out = pl.pallas_call(
        _expmap0_proj_body,
        out_shape=jax.ShapeDtypeStruct((n, d), jnp.float32),
    )(support_t.reshape(d, n))

    return out


# speedup vs baseline: 1.3448x; 1.0062x over previous
import functools

import jax
import jax.numpy as jnp
from jax import lax
from jax.experimental import pallas as pl
from jax.experimental.pallas import tpu as pltpu
from jax.experimental.pallas import tpu_sc as plsc

_MIN_NORM = 1e-15
_PROJ_EPS = 4e-3

_NC = 2
_NS = 16
_NW = _NC * _NS
_L = 16

_CE = 8000


def _artanh(v):
    v = jnp.clip(v, -1.0 + 1e-7, 1.0 - 1e-7)
    return 0.5 * (jnp.log1p(v) - jnp.log1p(-v))


def _logmap0_packed_body(x_ref, ei_ref, o_ref, oe_ref):
    x = x_ref[...]
    nrm = jnp.sqrt(jnp.sum(x * x, axis=1, keepdims=True))
    nrm = jnp.maximum(nrm, _MIN_NORM)
    scale = _artanh(nrm) / nrm
    xt_t = (x * scale).astype(jnp.bfloat16).T
    half = xt_t.shape[0] // 2
    lo = lax.bitcast_convert_type(xt_t[:half], jnp.uint16).astype(jnp.int32)
    hi = lax.bitcast_convert_type(xt_t[half:], jnp.uint16).astype(jnp.int32)
    o_ref[...] = lo | (hi << 16)
    oe_ref[...] = ei_ref[0, :] + ei_ref[1, :] * 16384


def _expmap0_proj_body(a_ref, o_ref):
    u = a_ref[...].T
    u_norm = jnp.maximum(
        jnp.sqrt(jnp.sum(u * u, axis=1, keepdims=True)), _MIN_NORM
    )
    gamma = jnp.tanh(u_norm) * u / u_norm
    g_norm = jnp.maximum(
        jnp.sqrt(jnp.sum(gamma * gamma, axis=1, keepdims=True)), _MIN_NORM
    )
    maxnorm = 1.0 - _PROJ_EPS
    projected = gamma / g_norm * maxnorm
    o_ref[...] = jnp.where(g_norm > maxnorm, projected, gamma)


def _make_spmm(n, d, e):
    half = d // 2
    ppt = half // _NW
    fpt = 2 * ppt
    pk_words = ppt * n
    acc_words = fpt * n
    n_chunks = e // _CE
    gpc = _CE // _L
    assert n <= 16384
    assert n_chunks % 2 == 0

    mesh = plsc.VectorSubcoreMesh(core_axis_name="c", subcore_axis_name="s")

    @functools.partial(
        pl.kernel,
        mesh=mesh,
        compiler_params=pltpu.CompilerParams(needs_layout_passes=False),
        out_type=jax.ShapeDtypeStruct((d * n,), jnp.float32),
        scratch_types=[
            pltpu.VMEM((pk_words,), jnp.int32),
            pltpu.VMEM((acc_words,), jnp.float32),
            pltpu.VMEM((_CE + _L,), jnp.int32),
            pltpu.VMEM((_CE + _L,), jnp.int32),
            pltpu.VMEM((_CE + _L,), jnp.float32),
            pltpu.VMEM((_CE + _L,), jnp.float32),
            pltpu.SemaphoreType.DMA,
            pltpu.SemaphoreType.DMA,
            pltpu.SemaphoreType.DMA,
        ],
    )
    def spmm(
        pk_hbm,
        pair_hbm,
        w_hbm,
        out_hbm,
        pk_v,
        acc_v,
        ep0,
        ep1,
        ew0,
        ew1,
        sem0,
        sem1,
        semx,
    ):
        wid = lax.axis_index("s") * _NC + lax.axis_index("c")
        pk_base = pl.multiple_of(wid * pk_words, 8)
        xt_copy = pltpu.async_copy(pk_hbm.at[pl.ds(pk_base, pk_words)], pk_v, semx)
        pltpu.async_copy(pair_hbm.at[pl.ds(0, _CE)], ep0.at[pl.ds(0, _CE)], sem0)
        pltpu.async_copy(w_hbm.at[pl.ds(0, _CE)], ew0.at[pl.ds(0, _CE)], sem0)

        zeros = jnp.zeros((_L,), jnp.float32)

        def zero_body(i, carry):
            acc_v[pl.ds(i * _L, _L)] = zeros
            return carry

        lax.fori_loop(0, acc_words // _L, zero_body, 0, unroll=8)
        xt_copy.wait()

        pbufs = (ep0, ep1)
        wbufs = (ew0, ew1)
        sems = (sem0, sem1)
        himask = jnp.int32(-65536)

        def outer_body(o, carry):
            for b in range(2):
                c = o * 2 + b
                ep = pbufs[b]
                ew = wbufs[b]
                pltpu.make_async_copy(
                    pair_hbm.at[pl.ds(0, _CE)], ep.at[pl.ds(0, _CE)], sems[b]
                ).wait()
                pltpu.make_async_copy(
                    w_hbm.at[pl.ds(0, _CE)], ew.at[pl.ds(0, _CE)], sems[b]
                ).wait()
                @pl.when(c + 1 < n_chunks)
                def _():
                    off = pl.multiple_of((c + 1) * _CE, 8)
                    pltpu.async_copy(
                        pair_hbm.at[pl.ds(off, _CE)],
                        pbufs[1 - b].at[pl.ds(0, _CE)],
                        sems[1 - b],
                    )
                    pltpu.async_copy(
                        w_hbm.at[pl.ds(off, _CE)],
                        wbufs[1 - b].at[pl.ds(0, _CE)],
                        sems[1 - b],
                    )

                @plsc.parallel_loop(0, gpc, unroll=10)
                def _(g):
                    gb = g * _L
                    pair = ep[pl.ds(gb, _L)]
                    s = pair & 16383
                    dd = lax.shift_right_logical(pair, 14)
                    w = ew[pl.ds(gb, _L)]
                    pk = [
                        plsc.load_gather(pk_v, [s + p * n]) for p in range(ppt)
                    ]
                    for p in range(ppt):
                        lo = plsc.bitcast(pk[p] << 16, jnp.float32)
                        hi = plsc.bitcast(pk[p] & himask, jnp.float32)
                        plsc.addupdate_scatter(acc_v, [dd + p * n], lo * w)
                        plsc.addupdate_scatter(
                            acc_v, [dd + (ppt + p) * n], hi * w
                        )
            return carry

        lax.fori_loop(0, n_chunks // 2, outer_body, 0)
        lo_base = pl.multiple_of(wid * ppt * n, 8)
        hi_base = pl.multiple_of((half + wid * ppt) * n, 8)
        pltpu.sync_copy(
            acc_v.at[pl.ds(0, ppt * n)], out_hbm.at[pl.ds(lo_base, ppt * n)]
        )
        pltpu.sync_copy(
            acc_v.at[pl.ds(ppt * n, ppt * n)],
            out_hbm.at[pl.ds(hi_base, ppt * n)],
        )

    return spmm


@jax.jit
def kernel(x, edge_index, edge_weight):
    n, d = x.shape
    e = edge_index.shape[1]

    pk_t, pair = pl.pallas_call(
        _logmap0_packed_body,
        out_shape=(
            jax.ShapeDtypeStruct((d // 2, n), jnp.int32),
            jax.ShapeDtypeStruct((e,), jnp.int32),
        ),
    )(x, edge_index)

    spmm = _make_spmm(n, d, e)
    support_t = spmm(pk_t.reshape(d // 2 * n), pair, edge_weight)

    out = pl.pallas_call(
        _expmap0_proj_body,
        out_shape=jax.ShapeDtypeStruct((n, d), jnp.float32),
    )(support_t.reshape(d, n))

    return out
